# baseline (speedup 1.0000x reference)
import functools

import jax
import jax.numpy as jnp
from jax import lax
from jax.experimental import pallas as pl
from jax.experimental.pallas import tpu as pltpu
from jax.experimental.pallas import tpu_sc as plsc

_NC = 2
_NS = 16
_L = 16
_NW = _NC * _NS

_TPAD = 1024


def _sc_gather_one(tbl, t_flat, n):
    chunk = n // _NW
    tlen = tbl.shape[0]
    mesh = plsc.VectorSubcoreMesh(core_axis_name="c", subcore_axis_name="s")

    @functools.partial(
        pl.kernel,
        out_type=jax.ShapeDtypeStruct((n,), jnp.float32),
        mesh=mesh,
        compiler_params=pltpu.CompilerParams(needs_layout_passes=False),
        scratch_types=[
            pltpu.VMEM((_TPAD,), jnp.float32),
            pltpu.VMEM((chunk,), jnp.int32),
            pltpu.VMEM((chunk,), jnp.float32),
            pltpu.SemaphoreType.DMA,
        ],
    )
    def gather_kernel(tbl_hbm, t_hbm, out_hbm, tbl_v, t_v, out_v, sem):
        wid = lax.axis_index("s") * _NC + lax.axis_index("c")
        base = wid * chunk
        cp1 = pltpu.async_copy(tbl_hbm, tbl_v.at[pl.ds(0, tlen)], sem)
        cp2 = pltpu.async_copy(t_hbm.at[pl.ds(base, chunk)], t_v, sem)
        cp1.wait()
        cp2.wait()
        for i in range(chunk // _L):
            idx = t_v[pl.ds(i * _L, _L)]
            out_v[pl.ds(i * _L, _L)] = plsc.load_gather(tbl_v, [idx])
        cp3 = pltpu.async_copy(out_v, out_hbm.at[pl.ds(base, chunk)], sem)
        cp3.wait()

    return gather_kernel(tbl, t_flat)


def _noise_body(g_ref, x_ref, nz_ref, out_ref, nz_out_ref):
    rows = x_ref.shape[0]
    g = g_ref[...].reshape(1, rows).T
    nz = nz_ref[...]
    out_ref[...] = jnp.sqrt(g) * x_ref[...] + jnp.sqrt(1.0 - g) * nz
    nz_out_ref[...] = nz


def _tc_noise_pair(g, x2, nz2, block_rows):
    n, d = x2.shape
    grid = (n // block_rows,)
    row_spec = pl.BlockSpec((block_rows, d), lambda i: (i, 0))
    g_spec = pl.BlockSpec((block_rows,), lambda i: (i,))
    arr = jax.ShapeDtypeStruct((n, d), jnp.float32)
    return pl.pallas_call(
        _noise_body,
        grid=grid,
        in_specs=[g_spec, row_spec, row_spec],
        out_specs=[row_spec, row_spec],
        out_shape=(arr, arr),
    )(g, x2, nz2)


def kernel(real, imag, real_gamma, imag_gamma, t, real_noise, imag_noise):
    b, s, d = real.shape
    n = b * s

    t_flat = t.reshape(n).astype(jnp.int32)

    gr = _sc_gather_one(real_gamma, t_flat, n)
    gi = _sc_gather_one(imag_gamma, t_flat, n)
    real_noisy, rn_out = _tc_noise_pair(
        gr, real.reshape(n, d), real_noise.reshape(n, d), block_rows=512)
    imag_noisy, inz_out = _tc_noise_pair(
        gi, imag.reshape(n, d), imag_noise.reshape(n, d), block_rows=512)
    return (real_noisy.reshape(b, s, d), rn_out.reshape(b, s, d),
            imag_noisy.reshape(b, s, d), inz_out.reshape(b, s, d), t)


# speedup vs baseline: 1.0539x; 1.0539x over previous
import functools

import jax
import jax.numpy as jnp
from jax import lax
from jax.experimental import pallas as pl
from jax.experimental.pallas import tpu as pltpu
from jax.experimental.pallas import tpu_sc as plsc

_NC = 2
_NS = 16
_L = 16
_NW = _NC * _NS

_TPAD = 1024


def _sc_gather(rg, ig, t_flat, n):
    chunk = n // _NS
    tlen = rg.shape[0]
    mesh = plsc.VectorSubcoreMesh(core_axis_name="c", subcore_axis_name="s",
                                  num_cores=1)

    @functools.partial(
        pl.kernel,
        out_type=(
            jax.ShapeDtypeStruct((n,), jnp.float32),
            jax.ShapeDtypeStruct((n,), jnp.float32),
        ),
        mesh=mesh,
        compiler_params=pltpu.CompilerParams(needs_layout_passes=False),
        scratch_types=[
            pltpu.VMEM((_TPAD,), jnp.float32),
            pltpu.VMEM((_TPAD,), jnp.float32),
            pltpu.VMEM((chunk,), jnp.int32),
            pltpu.VMEM((chunk,), jnp.float32),
            pltpu.VMEM((chunk,), jnp.float32),
            pltpu.SemaphoreType.DMA,
        ],
    )
    def gather_kernel(rg_hbm, ig_hbm, t_hbm, outr_hbm, outi_hbm,
                      rg_v, ig_v, t_v, outr_v, outi_v, sem):
        wid = lax.axis_index("s")
        base = wid * chunk
        cp1 = pltpu.async_copy(rg_hbm, rg_v.at[pl.ds(0, tlen)], sem)
        cp2 = pltpu.async_copy(ig_hbm, ig_v.at[pl.ds(0, tlen)], sem)
        cp3 = pltpu.async_copy(t_hbm.at[pl.ds(base, chunk)], t_v, sem)
        cp1.wait()
        cp2.wait()
        cp3.wait()
        for i in range(chunk // _L):
            idx = t_v[pl.ds(i * _L, _L)]
            outr_v[pl.ds(i * _L, _L)] = plsc.load_gather(rg_v, [idx])
            outi_v[pl.ds(i * _L, _L)] = plsc.load_gather(ig_v, [idx])
        cp4 = pltpu.async_copy(outr_v, outr_hbm.at[pl.ds(base, chunk)], sem)
        cp5 = pltpu.async_copy(outi_v, outi_hbm.at[pl.ds(base, chunk)], sem)
        cp4.wait()
        cp5.wait()

    return gather_kernel(rg, ig, t_flat)


def _noise_body(gr_ref, gi_ref, real_ref, rn_ref, imag_ref, inz_ref,
                outr_ref, outi_ref, rn_out_ref, inz_out_ref):
    rows = real_ref.shape[0]
    gr = gr_ref[...].reshape(1, rows).T
    rn = rn_ref[...]
    outr_ref[...] = jnp.sqrt(gr) * real_ref[...] + jnp.sqrt(1.0 - gr) * rn
    rn_out_ref[...] = rn
    gi = gi_ref[...].reshape(1, rows).T
    inz = inz_ref[...]
    outi_ref[...] = jnp.sqrt(gi) * imag_ref[...] + jnp.sqrt(1.0 - gi) * inz
    inz_out_ref[...] = inz


def _tc_noise(gr, gi, real2, rn2, imag2, inz2, block_rows):
    n, d = real2.shape
    grid = (n // block_rows,)
    row_spec = pl.BlockSpec((block_rows, d), lambda i: (i, 0))
    g_spec = pl.BlockSpec((block_rows,), lambda i: (i,))
    arr = jax.ShapeDtypeStruct((n, d), jnp.float32)
    return pl.pallas_call(
        _noise_body,
        grid=grid,
        in_specs=[g_spec, g_spec, row_spec, row_spec, row_spec, row_spec],
        out_specs=[row_spec, row_spec, row_spec, row_spec],
        out_shape=(arr, arr, arr, arr),
    )(gr, gi, real2, rn2, imag2, inz2)


def kernel(real, imag, real_gamma, imag_gamma, t, real_noise, imag_noise):
    b, s, d = real.shape
    n = b * s

    t_flat = t.reshape(n).astype(jnp.int32)

    gr, gi = _sc_gather(real_gamma, imag_gamma, t_flat, n)

    real_noisy, imag_noisy, rn_out, inz_out = _tc_noise(
        gr, gi,
        real.reshape(n, d), real_noise.reshape(n, d),
        imag.reshape(n, d), imag_noise.reshape(n, d),
        block_rows=512,
    )
    return (real_noisy.reshape(b, s, d), rn_out.reshape(b, s, d),
            imag_noisy.reshape(b, s, d), inz_out.reshape(b, s, d), t)


# speedup vs baseline: 1.0544x; 1.0005x over previous
import functools

import jax
import jax.numpy as jnp
from jax import lax
from jax.experimental import pallas as pl
from jax.experimental.pallas import tpu as pltpu
from jax.experimental.pallas import tpu_sc as plsc

_NS = 16
_L = 16

_TPAD = 1024


def _sc_gather(rg, ig, t_flat, n):
    chunk = n // _NS
    tlen = rg.shape[0]
    mesh = plsc.VectorSubcoreMesh(core_axis_name="c", subcore_axis_name="s",
                                  num_cores=1)

    @functools.partial(
        pl.kernel,
        out_type=(
            jax.ShapeDtypeStruct((n,), jnp.float32),
            jax.ShapeDtypeStruct((n,), jnp.float32),
        ),
        mesh=mesh,
        compiler_params=pltpu.CompilerParams(needs_layout_passes=False),
        scratch_types=[
            pltpu.VMEM((_TPAD,), jnp.float32),
            pltpu.VMEM((_TPAD,), jnp.float32),
            pltpu.VMEM((chunk,), jnp.int32),
            pltpu.VMEM((chunk,), jnp.float32),
            pltpu.VMEM((chunk,), jnp.float32),
            pltpu.SemaphoreType.DMA,
        ],
    )
    def gather_kernel(rg_hbm, ig_hbm, t_hbm, outr_hbm, outi_hbm,
                      rg_v, ig_v, t_v, outr_v, outi_v, sem):
        wid = lax.axis_index("s")
        base = wid * chunk
        cp1 = pltpu.async_copy(rg_hbm, rg_v.at[pl.ds(0, tlen)], sem)
        cp2 = pltpu.async_copy(ig_hbm, ig_v.at[pl.ds(0, tlen)], sem)
        cp3 = pltpu.async_copy(t_hbm.at[pl.ds(base, chunk)], t_v, sem)
        cp1.wait()
        cp2.wait()
        cp3.wait()
        for i in range(chunk // _L):
            idx = t_v[pl.ds(i * _L, _L)]
            outr_v[pl.ds(i * _L, _L)] = plsc.load_gather(rg_v, [idx])
            outi_v[pl.ds(i * _L, _L)] = plsc.load_gather(ig_v, [idx])
        cp4 = pltpu.async_copy(outr_v, outr_hbm.at[pl.ds(base, chunk)], sem)
        cp5 = pltpu.async_copy(outi_v, outi_hbm.at[pl.ds(base, chunk)], sem)
        cp4.wait()
        cp5.wait()

    return gather_kernel(rg, ig, t_flat)


def _noise_body(gr_ref, gi_ref, real_ref, rn_ref, imag_ref, inz_ref,
                outr_ref, outi_ref, rn_out_ref, inz_out_ref):
    rows = real_ref.shape[0]
    gr = gr_ref[...].reshape(1, rows).T
    rn = rn_ref[...]
    outr_ref[...] = jnp.sqrt(gr) * real_ref[...] + jnp.sqrt(1.0 - gr) * rn
    rn_out_ref[...] = rn
    gi = gi_ref[...].reshape(1, rows).T
    inz = inz_ref[...]
    outi_ref[...] = jnp.sqrt(gi) * imag_ref[...] + jnp.sqrt(1.0 - gi) * inz
    inz_out_ref[...] = inz


def _tc_noise(gr, gi, real2, rn2, imag2, inz2, block_rows):
    n, d = real2.shape
    grid = (n // block_rows,)
    row_spec = pl.BlockSpec((block_rows, d), lambda i: (i, 0))
    g_spec = pl.BlockSpec((block_rows,), lambda i: (i,))
    arr = jax.ShapeDtypeStruct((n, d), jnp.float32)
    return pl.pallas_call(
        _noise_body,
        grid=grid,
        in_specs=[g_spec, g_spec, row_spec, row_spec, row_spec, row_spec],
        out_specs=[row_spec, row_spec, row_spec, row_spec],
        out_shape=(arr, arr, arr, arr),
    )(gr, gi, real2, rn2, imag2, inz2)


def kernel(real, imag, real_gamma, imag_gamma, t, real_noise, imag_noise):
    b, s, d = real.shape
    n = b * s

    t_flat = t.reshape(n).astype(jnp.int32)

    gr, gi = _sc_gather(real_gamma, imag_gamma, t_flat, n)

    real_noisy, imag_noisy, rn_out, inz_out = _tc_noise(
        gr, gi,
        real.reshape(n, d), real_noise.reshape(n, d),
        imag.reshape(n, d), imag_noise.reshape(n, d),
        block_rows=512,
    )
    return (real_noisy.reshape(b, s, d), rn_out.reshape(b, s, d),
            imag_noisy.reshape(b, s, d), inz_out.reshape(b, s, d), t)
